# baseline (speedup 1.0000x reference)
import functools

import jax
import jax.numpy as jnp
from jax import lax
from jax.experimental import pallas as pl
from jax.experimental.pallas import tpu as pltpu
from jax.experimental.pallas import tpu_sc as plsc

BATCH = 4096
HIST = 200
EMBED_DIM = 32

_info = plsc.get_sparse_core_info()
_NC, _NS = _info.num_cores, _info.num_subcores
_NW = _NC * _NS
_BW = BATCH // _NW


def _make_gather(V, D):
    mesh = plsc.VectorSubcoreMesh(core_axis_name="c", subcore_axis_name="s")

    @functools.partial(
        pl.kernel,
        mesh=mesh,
        compiler_params=pltpu.CompilerParams(
            use_tc_tiling_on_sc=False, needs_layout_passes=False),
        out_type=jax.ShapeDtypeStruct((HIST, D // 8, _NW, 8, _BW),
                                      jnp.float32),
        scratch_types=[
            pltpu.VMEM((_BW,), jnp.int32),
            pltpu.VMEM((_BW,), jnp.int32),
            pltpu.VMEM((_BW, D), jnp.float32),
            pltpu.VMEM((_BW, D), jnp.float32),
            pltpu.VMEM((D, _BW), jnp.float32),
            pltpu.VMEM((D, _BW), jnp.float32),
            pltpu.SemaphoreType.DMA,
            pltpu.SemaphoreType.DMA,
            pltpu.SemaphoreType.DMA,
            pltpu.SemaphoreType.DMA,
            pltpu.SemaphoreType.DMA,
            pltpu.SemaphoreType.DMA,
        ],
    )
    def gather_kernel(table_hbm, idxt_hbm, out_hbm, idx0, idx1, rows0, rows1,
                      trows0, trows1, isem0, isem1, gsem0, gsem1, ssem0,
                      ssem1):
        wid = lax.axis_index("s") * _NC + lax.axis_index("c")
        bw = wid * _BW
        idxv = (idx0, idx1)
        rows = (rows0, rows1)
        trows = (trows0, trows1)
        isem = (isem0, isem1)
        gsem = (gsem0, gsem1)
        ssem = (ssem0, ssem1)

        def idx_of(h, p):
            return pltpu.make_async_copy(
                idxt_hbm.at[h, pl.ds(bw, _BW)], idxv[p], isem[p])

        def gather_of(p):
            return pltpu.make_async_copy(
                table_hbm.at[idxv[p]], rows[p], gsem[p])

        def store_pieces(h, p):
            return [
                pltpu.make_async_copy(
                    trows[p].at[pl.ds(8 * k, 8), :],
                    out_hbm.at[h, k, wid, :, :], ssem[p])
                for k in range(D // 8)
            ]

        def store_start(h, p):
            for c in store_pieces(h, p):
                c.start()

        def store_wait(h, p):
            for c in store_pieces(h, p):
                c.wait()

        bvecs = [lax.iota(jnp.int32, 16) + 16 * j for j in range(_BW // 16)]

        def transpose_rows(p):
            for d in range(D):
                dvec = jnp.full((16,), d, jnp.int32)
                for j in range(_BW // 16):
                    trows[p][d, pl.ds(16 * j, 16)] = plsc.load_gather(
                        rows[p], [bvecs[j], dvec])

        idx_of(0, 0).start()
        idx_of(0, 0).wait()
        gather_of(0).start()
        idx_of(1, 1).start()

        def outer(g, carry):
            for b in range(2):
                h = 2 * g + b
                nb = 1 - b
                gather_of(b).wait()

                @pl.when(h + 1 < HIST)
                def _():
                    idx_of(h + 1, nb).wait()
                    gather_of(nb).start()

                @pl.when(h + 2 < HIST)
                def _():
                    idx_of(h + 2, b).start()

                @pl.when(h >= 2)
                def _():
                    store_wait(h - 2, b)

                transpose_rows(b)
                store_start(h, b)
            return carry

        lax.fori_loop(0, HIST // 2, outer, 0)
        store_wait(HIST - 2, 0)
        store_wait(HIST - 1, 1)

    return gather_kernel


def kernel(input, table):
    idx_t = input.T.astype(jnp.int32)
    V, D = table.shape
    out5 = _make_gather(V, D)(table, idx_t)
    return out5.transpose(2, 4, 0, 1, 3).reshape(BATCH, HIST, D)


# speedup vs baseline: 1.2015x; 1.2015x over previous
import functools

import jax
import jax.numpy as jnp
from jax import lax
from jax.experimental import pallas as pl
from jax.experimental.pallas import tpu as pltpu
from jax.experimental.pallas import tpu_sc as plsc

BATCH = 4096
HIST = 200
EMBED_DIM = 32

_info = plsc.get_sparse_core_info()
_NC, _NS = _info.num_cores, _info.num_subcores
_NW = _NC * _NS
_BW = BATCH // _NW
_HS = 4
_NG = HIST // _HS


def _make_gather(V, D):
    mesh = plsc.VectorSubcoreMesh(core_axis_name="c", subcore_axis_name="s")

    @functools.partial(
        pl.kernel,
        mesh=mesh,
        compiler_params=pltpu.CompilerParams(
            use_tc_tiling_on_sc=False, needs_layout_passes=False),
        out_type=jax.ShapeDtypeStruct((HIST, D // 8, _NW, 8, _BW),
                                      jnp.float32),
        scratch_types=[
            pltpu.VMEM((_HS, _BW), jnp.int32),
            pltpu.VMEM((_HS, _BW), jnp.int32),
            pltpu.VMEM((_HS, _BW, D), jnp.float32),
            pltpu.VMEM((_HS, _BW, D), jnp.float32),
            pltpu.VMEM((_HS, D, _BW), jnp.float32),
            pltpu.VMEM((_HS, D, _BW), jnp.float32),
            pltpu.SemaphoreType.DMA,
            pltpu.SemaphoreType.DMA,
            pltpu.SemaphoreType.DMA,
            pltpu.SemaphoreType.DMA,
            pltpu.SemaphoreType.DMA,
            pltpu.SemaphoreType.DMA,
        ],
    )
    def gather_kernel(table_hbm, idxt_hbm, out_hbm, idx0, idx1, rows0, rows1,
                      trows0, trows1, isem0, isem1, gsem0, gsem1, ssem0,
                      ssem1):
        wid = lax.axis_index("s") * _NC + lax.axis_index("c")
        bw = wid * _BW
        idxv = (idx0, idx1)
        rows = (rows0, rows1)
        trows = (trows0, trows1)
        isem = (isem0, isem1)
        gsem = (gsem0, gsem1)
        ssem = (ssem0, ssem1)

        def idx_of(g, p):
            return pltpu.make_async_copy(
                idxt_hbm.at[pl.ds(g * _HS, _HS), pl.ds(bw, _BW)], idxv[p],
                isem[p])

        def gathers_of(p):
            return [
                pltpu.make_async_copy(
                    table_hbm.at[idxv[p].at[r]], rows[p].at[r], gsem[p])
                for r in range(_HS)
            ]

        def stores_of(g, p):
            return [
                pltpu.make_async_copy(
                    trows[p].at[:, pl.ds(8 * k, 8), :],
                    out_hbm.at[pl.ds(g * _HS, _HS), k, wid, :, :], ssem[p])
                for k in range(D // 8)
            ]

        bvecs = [lax.iota(jnp.int32, 16) + 16 * j for j in range(_BW // 16)]

        def transpose_group(p):
            for r in range(_HS):
                src = rows[p].at[r]
                for d in range(D):
                    dvec = jnp.full((16,), d, jnp.int32)
                    vals = [
                        plsc.load_gather(src, [bvecs[j], dvec])
                        for j in range(_BW // 16)
                    ]
                    for j in range(_BW // 16):
                        trows[p][r, d, pl.ds(16 * j, 16)] = vals[j]

        idx_of(0, 0).start()
        idx_of(0, 0).wait()
        for c in gathers_of(0):
            c.start()
        idx_of(1, 1).start()

        def outer(gg, carry):
            for b in range(2):
                g = 2 * gg + b
                nb = 1 - b
                for c in gathers_of(b):
                    c.wait()

                @pl.when(g + 1 < _NG)
                def _():
                    idx_of(g + 1, nb).wait()
                    for c in gathers_of(nb):
                        c.start()

                @pl.when(g + 2 < _NG)
                def _():
                    idx_of(g + 2, b).start()

                @pl.when(g >= 2)
                def _():
                    for c in stores_of(g - 2, b):
                        c.wait()

                transpose_group(b)
                for c in stores_of(g, b):
                    c.start()
            return carry

        lax.fori_loop(0, _NG // 2, outer, 0)
        for c in stores_of(_NG - 2, 0):
            c.wait()
        for c in stores_of(_NG - 1, 1):
            c.wait()

    return gather_kernel


def kernel(input, table):
    idx_t = input.T.astype(jnp.int32)
    V, D = table.shape
    out5 = _make_gather(V, D)(table, idx_t)
    return out5.transpose(2, 4, 0, 1, 3).reshape(BATCH, HIST, D)


# speedup vs baseline: 1.6004x; 1.3320x over previous
import functools

import jax
import jax.numpy as jnp
from jax import lax
from jax.experimental import pallas as pl
from jax.experimental.pallas import tpu as pltpu
from jax.experimental.pallas import tpu_sc as plsc

BATCH = 4096
HIST = 200
EMBED_DIM = 32

_info = plsc.get_sparse_core_info()
_NC, _NS = _info.num_cores, _info.num_subcores
_NW = _NC * _NS
_BW = BATCH // _NW
_HS = 4
_NG = HIST // _HS
_BWP = _BW + 1


def _make_gather(V, D):
    mesh = plsc.VectorSubcoreMesh(core_axis_name="c", subcore_axis_name="s")

    @functools.partial(
        pl.kernel,
        mesh=mesh,
        compiler_params=pltpu.CompilerParams(
            use_tc_tiling_on_sc=False, needs_layout_passes=False),
        out_type=jax.ShapeDtypeStruct((HIST, D // 8, _NW, 8, _BW),
                                      jnp.float32),
        scratch_types=[
            pltpu.VMEM((_HS, _BW), jnp.int32),
            pltpu.VMEM((_HS, _BW), jnp.int32),
            pltpu.VMEM((_HS, _BW, D), jnp.float32),
            pltpu.VMEM((_HS, _BW, D), jnp.float32),
            pltpu.VMEM((_HS, D, _BWP), jnp.float32),
            pltpu.VMEM((_HS, D, _BWP), jnp.float32),
            pltpu.SemaphoreType.DMA,
            pltpu.SemaphoreType.DMA,
            pltpu.SemaphoreType.DMA,
            pltpu.SemaphoreType.DMA,
            pltpu.SemaphoreType.DMA,
            pltpu.SemaphoreType.DMA,
        ],
    )
    def gather_kernel(table_hbm, idxt_hbm, out_hbm, idx0, idx1, rows0, rows1,
                      trows0, trows1, isem0, isem1, gsem0, gsem1, ssem0,
                      ssem1):
        wid = lax.axis_index("s") * _NC + lax.axis_index("c")
        bw = wid * _BW
        idxv = (idx0, idx1)
        rows = (rows0, rows1)
        trows = (trows0, trows1)
        isem = (isem0, isem1)
        gsem = (gsem0, gsem1)
        ssem = (ssem0, ssem1)

        def idx_of(g, p):
            return pltpu.make_async_copy(
                idxt_hbm.at[pl.ds(g * _HS, _HS), pl.ds(bw, _BW)], idxv[p],
                isem[p])

        def gathers_of(p):
            return [
                pltpu.make_async_copy(
                    table_hbm.at[idxv[p].at[r]], rows[p].at[r], gsem[p])
                for r in range(_HS)
            ]

        def stores_of(g, p):
            return [
                pltpu.make_async_copy(
                    trows[p].at[:, pl.ds(8 * k, 8), pl.ds(0, _BW)],
                    out_hbm.at[pl.ds(g * _HS, _HS), k, wid, :, :], ssem[p])
                for k in range(D // 8)
            ]

        dvecs = [lax.iota(jnp.int32, 16) + 16 * dh for dh in range(D // 16)]

        def transpose_group(p):
            for r in range(_HS):
                src = rows[p].at[r]
                dst = trows[p].at[r]
                for bb in range(0, _BW, 4):
                    vals = [
                        src[b, pl.ds(16 * dh, 16)]
                        for b in range(bb, bb + 4)
                        for dh in range(D // 16)
                    ]
                    i = 0
                    for b in range(bb, bb + 4):
                        bs = jnp.full((16,), b, jnp.int32)
                        for dh in range(D // 16):
                            plsc.store_scatter(dst, [dvecs[dh], bs], vals[i])
                            i += 1

        idx_of(0, 0).start()
        idx_of(0, 0).wait()
        for c in gathers_of(0):
            c.start()
        idx_of(1, 1).start()

        def outer(gg, carry):
            for b in range(2):
                g = 2 * gg + b
                nb = 1 - b
                for c in gathers_of(b):
                    c.wait()

                @pl.when(g + 1 < _NG)
                def _():
                    idx_of(g + 1, nb).wait()
                    for c in gathers_of(nb):
                        c.start()

                @pl.when(g + 2 < _NG)
                def _():
                    idx_of(g + 2, b).start()

                @pl.when(g >= 2)
                def _():
                    for c in stores_of(g - 2, b):
                        c.wait()

                transpose_group(b)
                for c in stores_of(g, b):
                    c.start()
            return carry

        lax.fori_loop(0, _NG // 2, outer, 0)
        for c in stores_of(_NG - 2, 0):
            c.wait()
        for c in stores_of(_NG - 1, 1):
            c.wait()

    return gather_kernel


def kernel(input, table):
    idx_t = input.T.astype(jnp.int32)
    V, D = table.shape
    out5 = _make_gather(V, D)(table, idx_t)
    return out5.transpose(2, 4, 0, 1, 3).reshape(BATCH, HIST, D)
